# Initial kernel scaffold; baseline (speedup 1.0000x reference)
import jax
import jax.numpy as jnp
from jax.experimental import pallas as pl


def kernel(pairs, features):
    raise NotImplementedError("write your pallas kernel here")


# speedup vs baseline: 4.3331x; 4.3331x over previous
import functools

import jax
import jax.numpy as jnp
from jax import lax
from jax.experimental import pallas as pl
from jax.experimental.pallas import tpu as pltpu
from jax.experimental.pallas import tpu_sc as plsc

GAMMA = 1.0

NC = 2
NS = 16
NW = NC * NS
LANES = 16


def _sc_body(nsteps, g, chunk, p_valid, d,
             idx_hbm, feat_hbm, out_hbm, idxl_v, idxr_v, idxfl_v, idxfr_v,
             rows_l, rows_r, rows_fl, rows_fr, out_stage, sem):
  wid = lax.axis_index("s") * NC + lax.axis_index("c")
  base = wid * chunk

  p_pad = NW * chunk
  idx_bufs = (idxl_v, idxr_v, idxfl_v, idxfr_v)
  for c in range(4):
    pltpu.sync_copy(idx_hbm.at[pl.ds(c * p_pad + base, chunk)], idx_bufs[c])

  nq = d // LANES
  row_bufs = (rows_l, rows_r, rows_fl, rows_fr)
  lane = lax.iota(jnp.int32, LANES)
  rots = [(lane + s) & (LANES - 1) for s in (8, 4, 2, 1)]

  dnums = lax.GatherDimensionNumbers(
      offset_dims=(), collapsed_slice_dims=(0,), start_index_map=(0,))

  def hsum(x):
    for perm in rots:
      rot = lax.gather(x, perm[:, None], dnums, slice_sizes=(1,),
                       mode=lax.GatherScatterMode.PROMISE_IN_BOUNDS)
      x = x + rot
    return x

  unroll = 4

  def step(t, acc):
    copies = []
    for c in range(4):
      copies.append(
          pltpu.async_copy(feat_hbm.at[idx_bufs[c].at[pl.ds(t * g, g)]],
                           row_bufs[c], sem))
    for cp in copies:
      cp.wait()

    def pair_group(pg, acc2):
      for k in range(unroll):
        i = pg * unroll + k
        u = None
        v = None
        for q in range(nq):
          sl = pl.ds(q * LANES, LANES)
          lv = rows_l[i, sl]
          rv = rows_r[i, sl]
          flv = rows_fl[i, sl]
          frv = rows_fr[i, sl]
          a = jnp.abs(lv - rv)
          du = a - jnp.abs(lv - frv)
          dv = a - jnp.abs(flv - rv)
          u = du if u is None else u + du
          v = dv if v is None else v + dv
        su = hsum(u)
        sv = hsum(v)
        contrib = (jnp.maximum(GAMMA + su, 0.0) +
                   jnp.maximum(GAMMA + sv, 0.0))
        valid = (base + t * g + i) < p_valid
        acc2 = acc2 + jnp.where(valid, contrib, 0.0)
      return acc2

    return lax.fori_loop(0, g // unroll, pair_group, acc)

  acc = lax.fori_loop(0, nsteps, step, jnp.zeros((LANES,), jnp.float32))

  out_stage[...] = jnp.where(lane == 0, acc, 0.0)
  pltpu.sync_copy(out_stage, out_hbm.at[wid])


def _build_sc_call(p_valid, v_rows, d, g):
  chunk = ((p_valid + NW * g - 1) // (NW * g)) * g
  nsteps = chunk // g
  mesh = plsc.VectorSubcoreMesh(core_axis_name="c", subcore_axis_name="s")
  body = functools.partial(_sc_body, nsteps, g, chunk, p_valid, d)
  return chunk, pl.kernel(
      body,
      out_type=jax.ShapeDtypeStruct((NW, LANES), jnp.float32),
      mesh=mesh,
      scratch_types=[
          pltpu.VMEM((chunk,), jnp.int32),
          pltpu.VMEM((chunk,), jnp.int32),
          pltpu.VMEM((chunk,), jnp.int32),
          pltpu.VMEM((chunk,), jnp.int32),
          pltpu.VMEM((g, d), jnp.float32),
          pltpu.VMEM((g, d), jnp.float32),
          pltpu.VMEM((g, d), jnp.float32),
          pltpu.VMEM((g, d), jnp.float32),
          pltpu.VMEM((LANES,), jnp.float32),
          pltpu.SemaphoreType.DMA,
      ],
  )


def kernel(pairs, features):
  p, _ = pairs.shape
  v_rows, d = features.shape
  g = 64
  chunk, call = _build_sc_call(p, v_rows, d, g)
  p_pad = NW * chunk
  idx = jnp.zeros((4, p_pad), jnp.int32).at[:, :p].set(pairs.T).reshape(-1)
  partials = call(idx, features)
  return jnp.sum(partials) / v_rows
